# baseline (speedup 1.0000x reference)
import jax
import jax.numpy as jnp
from jax import lax
from jax.experimental import pallas as pl
from jax.experimental.pallas import tpu as pltpu
from jax.experimental.pallas import tpu_sc as plsc

NN = 10000
EE = 160000
DD = 256
NP = 10240
EP = 163840
RB = 1024
NBLK = NP // RB
PAD_IDX = NP - 1
HALF = 128
CHK = 128
NCHK = EP // 16 // CHK

_f32 = jnp.float32


def _deg_body(dst_hbm, z1_hbm, d0_hbm, d1_hbm, dst_v, ones_v, acc_sp):
    c = lax.axis_index("c")
    s = lax.axis_index("s")
    wid = s * 2 + c

    @pl.when(s == 0)
    def _():
        pltpu.sync_copy(z1_hbm, acc_sp)

    plsc.subcore_barrier()
    pltpu.sync_copy(dst_hbm.at[wid], dst_v)
    for k in range(8):
        ones_v[pl.ds(k * 16, 16)] = jnp.ones((16,), _f32)

    def body(j, carry):
        pltpu.sync_copy(ones_v, acc_sp.at[dst_v.at[j]], add=True)
        return carry

    lax.fori_loop(0, 40, body, 0)
    plsc.subcore_barrier()
    rows = pl.ds(s * (NP // 16), NP // 16)

    @pl.when(c == 0)
    def _():
        pltpu.sync_copy(acc_sp.at[rows], d0_hbm.at[rows])

    @pl.when(c == 1)
    def _():
        pltpu.sync_copy(acc_sp.at[rows], d1_hbm.at[rows])


def _deg_call(dst32, z1):
    f = pl.kernel(
        _deg_body,
        out_type=[jax.ShapeDtypeStruct((NP,), _f32),
                  jax.ShapeDtypeStruct((NP,), _f32)],
        mesh=plsc.VectorSubcoreMesh(core_axis_name="c", subcore_axis_name="s", num_cores=2, num_subcores=16),
        scratch_types=[pltpu.VMEM((40, 128), jnp.int32),
                       pltpu.VMEM((128,), _f32),
                       pltpu.VMEM_SHARED((NP,), _f32)],
    )
    return f(dst32, z1)


def _mm1_body(x_ref, w1_ref, d0_ref, d1_ref, h1_ref, g0_ref, g1_ref, dinv_ref):
    deg = d0_ref[...] + d1_ref[...] + 1.0
    dinv = lax.rsqrt(deg)
    h1 = jnp.dot(x_ref[...], w1_ref[...], preferred_element_type=_f32)
    g = dinv * h1
    h1_ref[...] = h1
    g0_ref[...] = g[:, :HALF]
    g1_ref[...] = g[:, HALF:]
    dinv_ref[...] = dinv


def _mm1_call(xp, w1, d0, d1):
    return pl.pallas_call(
        _mm1_body,
        grid=(NBLK,),
        in_specs=[
            pl.BlockSpec((RB, DD), lambda i: (i, 0)),
            pl.BlockSpec((DD, DD), lambda i: (0, 0)),
            pl.BlockSpec((RB, 1), lambda i: (i, 0)),
            pl.BlockSpec((RB, 1), lambda i: (i, 0)),
        ],
        out_specs=[
            pl.BlockSpec((RB, DD), lambda i: (i, 0)),
            pl.BlockSpec((RB, HALF), lambda i: (i, 0)),
            pl.BlockSpec((RB, HALF), lambda i: (i, 0)),
            pl.BlockSpec((RB, 1), lambda i: (i, 0)),
        ],
        out_shape=[
            jax.ShapeDtypeStruct((NP, DD), _f32),
            jax.ShapeDtypeStruct((NP, HALF), _f32),
            jax.ShapeDtypeStruct((NP, HALF), _f32),
            jax.ShapeDtypeStruct((NP, 1), _f32),
        ],
        compiler_params=pltpu.CompilerParams(
            dimension_semantics=("arbitrary",)),
    )(xp, w1, d0, d1)


def _mp_body(g0_hbm, g1_hbm, src_hbm, dst_hbm, z2_hbm, s0_hbm, s1_hbm,
             src_v, dst_v, rows_a, acc_sp, sem_a):
    c = lax.axis_index("c")
    s = lax.axis_index("s")

    @pl.when(s == 0)
    def _():
        pltpu.sync_copy(z2_hbm, acc_sp)

    plsc.subcore_barrier()
    pltpu.sync_copy(src_hbm.at[s], src_v)
    pltpu.sync_copy(dst_hbm.at[s], dst_v)

    def mk(g_hbm):
        def body(j, carry):
            pltpu.async_copy(g_hbm.at[src_v.at[j]], rows_a, sem_a).wait()
            pltpu.sync_copy(rows_a, acc_sp.at[dst_v.at[j]], add=True)
            return carry
        return body

    @pl.when(c == 0)
    def _():
        lax.fori_loop(0, NCHK, mk(g0_hbm), 0)

    @pl.when(c == 1)
    def _():
        lax.fori_loop(0, NCHK, mk(g1_hbm), 0)

    plsc.subcore_barrier()
    rows = pl.ds(s * (NP // 16), NP // 16)

    @pl.when(c == 0)
    def _():
        pltpu.sync_copy(acc_sp.at[rows], s0_hbm.at[rows])

    @pl.when(c == 1)
    def _():
        pltpu.sync_copy(acc_sp.at[rows], s1_hbm.at[rows])


def _mp_call(g0, g1, src16, dst16, z2):
    f = pl.kernel(
        _mp_body,
        out_type=[jax.ShapeDtypeStruct((NP, HALF), _f32),
                  jax.ShapeDtypeStruct((NP, HALF), _f32)],
        mesh=plsc.VectorSubcoreMesh(core_axis_name="c", subcore_axis_name="s", num_cores=2, num_subcores=16),
        scratch_types=[pltpu.VMEM((NCHK, CHK), jnp.int32),
                       pltpu.VMEM((NCHK, CHK), jnp.int32),
                       pltpu.VMEM((CHK, HALF), _f32),
                       pltpu.VMEM_SHARED((NP, HALF), _f32),
                       pltpu.SemaphoreType.DMA],
    )
    return f(g0, g1, src16, dst16, z2)


def _gi_body(s0_ref, s1_ref, h1_ref, dinv_ref, b1_ref, wih_ref, bih_ref,
             gi_ref):
    dinv = dinv_ref[...]
    scat = jnp.concatenate([s0_ref[...], s1_ref[...]], axis=1).astype(_f32)
    out1 = jnp.maximum(
        dinv * scat + (dinv * dinv) * h1_ref[...] + b1_ref[...], 0.0)
    gi_ref[...] = (jnp.dot(out1, wih_ref[...], preferred_element_type=_f32)
                   + bih_ref[...])


def _gi_call(s0, s1, h1, dinv, b1r, wihT, bihr):
    return pl.pallas_call(
        _gi_body,
        grid=(NBLK,),
        in_specs=[
            pl.BlockSpec((RB, HALF), lambda i: (i, 0)),
            pl.BlockSpec((RB, HALF), lambda i: (i, 0)),
            pl.BlockSpec((RB, DD), lambda i: (i, 0)),
            pl.BlockSpec((RB, 1), lambda i: (i, 0)),
            pl.BlockSpec((1, DD), lambda i: (0, 0)),
            pl.BlockSpec((DD, 3 * DD), lambda i: (0, 0)),
            pl.BlockSpec((1, 3 * DD), lambda i: (0, 0)),
        ],
        out_specs=pl.BlockSpec((RB, 3 * DD), lambda i: (i, 0)),
        out_shape=jax.ShapeDtypeStruct((NP, 3 * DD), _f32),
        compiler_params=pltpu.CompilerParams(
            dimension_semantics=("arbitrary",)),
    )(s0, s1, h1, dinv, b1r, wihT, bihr)


def _gru_body(gi_ref, whh_ref, bhn_ref, w2_ref, dinv_ref, q_ref,
              h_scr, ys_scr):
    i = pl.program_id(0)

    @pl.when(i == 0)
    def _():
        h_scr[...] = jnp.zeros((1, DD), _f32)

    whh = whh_ref[...]
    bhn = bhn_ref[...]

    def step(t, h):
        gi = gi_ref[pl.ds(t, 1), :]
        gh = jnp.dot(h.astype(jnp.bfloat16), whh, preferred_element_type=_f32)
        r = 0.5 * jnp.tanh(0.5 * (gi[:, :DD] + gh[:, :DD])) + 0.5
        z = 0.5 * jnp.tanh(0.5 * (gi[:, DD:2 * DD] + gh[:, DD:2 * DD])) + 0.5
        ng = jnp.tanh(gi[:, 2 * DD:] + r * (gh[:, 2 * DD:] + bhn))
        h = ng + z * (h - ng)
        ys_scr[pl.ds(t, 1), :] = h
        return h

    h = lax.fori_loop(0, RB, step, h_scr[...])
    h_scr[...] = h
    yb = jnp.dot(ys_scr[...], w2_ref[...], preferred_element_type=_f32)
    q_ref[...] = dinv_ref[...] * yb


def _gru_call(gi, whhT_bf, bhn, w2p, dinv):
    return pl.pallas_call(
        _gru_body,
        grid=(NBLK,),
        in_specs=[
            pl.BlockSpec((RB, 3 * DD), lambda i: (i, 0)),
            pl.BlockSpec((DD, 3 * DD), lambda i: (0, 0)),
            pl.BlockSpec((1, DD), lambda i: (0, 0)),
            pl.BlockSpec((DD, HALF), lambda i: (0, 0)),
            pl.BlockSpec((RB, 1), lambda i: (i, 0)),
        ],
        out_specs=pl.BlockSpec((RB, HALF), lambda i: (i, 0)),
        out_shape=jax.ShapeDtypeStruct((NP, HALF), _f32),
        scratch_shapes=[pltpu.VMEM((1, DD), _f32),
                        pltpu.VMEM((RB, DD), _f32)],
        compiler_params=pltpu.CompilerParams(
            dimension_semantics=("arbitrary",)),
    )(gi, whhT_bf, bhn, w2p, dinv)


def _fin_body(q_hbm, src_hbm, dst_hbm, dinv_hbm, b2_hbm, z1_hbm, out_hbm,
              src_v, dst_v, vals_v, accv, qv, dv, outv, b2v, acc_sp, sem):
    c = lax.axis_index("c")
    s = lax.axis_index("s")

    @pl.when(c == 0)
    def _():
        @pl.when(s == 0)
        def _():
            pltpu.sync_copy(z1_hbm, acc_sp)

        plsc.subcore_barrier()
        pltpu.sync_copy(src_hbm.at[s], src_v)
        pltpu.sync_copy(dst_hbm.at[s], dst_v)

        def body(j, carry):
            pltpu.async_copy(q_hbm.at[src_v.at[j]], vals_v, sem).wait()
            pltpu.sync_copy(vals_v, acc_sp.at[dst_v.at[j]], add=True)
            return carry

        lax.fori_loop(0, 80, body, 0)
        plsc.subcore_barrier()
        rows = pl.ds(s * (NP // 16), NP // 16)
        pltpu.sync_copy(acc_sp.at[rows], accv)
        pltpu.sync_copy(q_hbm.at[rows], qv)
        pltpu.sync_copy(dinv_hbm.at[rows], dv)
        pltpu.sync_copy(b2_hbm, b2v)
        b2x = b2v[...]
        for k in range(NP // 16 // 16):
            sl = pl.ds(k * 16, 16)
            outv[sl] = dv[sl] * (accv[sl] + qv[sl]) + b2x
        pltpu.sync_copy(outv, out_hbm.at[rows])


def _fin_call(q1d, src16, dst16, dinv1d, b2v16, z1):
    f = pl.kernel(
        _fin_body,
        out_type=jax.ShapeDtypeStruct((NP,), _f32),
        mesh=plsc.VectorSubcoreMesh(core_axis_name="c", subcore_axis_name="s", num_cores=2, num_subcores=16),
        scratch_types=[pltpu.VMEM((80, 128), jnp.int32),
                       pltpu.VMEM((80, 128), jnp.int32),
                       pltpu.VMEM((128,), _f32),
                       pltpu.VMEM((NP // 16,), _f32),
                       pltpu.VMEM((NP // 16,), _f32),
                       pltpu.VMEM((NP // 16,), _f32),
                       pltpu.VMEM((NP // 16,), _f32),
                       pltpu.VMEM((16,), _f32),
                       pltpu.VMEM_SHARED((NP,), _f32),
                       pltpu.SemaphoreType.DMA],
    )
    return f(q1d, src16, dst16, dinv1d, b2v16, z1)


def kernel(x, edge_index, W1, b1, W_ih, W_hh, b_ih, b_hh, W2, b2):
    src = edge_index[0].astype(jnp.int32)
    dst = edge_index[1].astype(jnp.int32)
    pad = jnp.full((EP - EE,), PAD_IDX, jnp.int32)
    src_p = jnp.concatenate([src, pad])
    dst_p = jnp.concatenate([dst, pad])
    src32 = src_p.reshape(32, 40, 128)
    dst32 = dst_p.reshape(32, 40, 128)
    src16 = src_p.reshape(16, 80, 128)
    dst16 = dst_p.reshape(16, 80, 128)

    xp = jnp.zeros((NP, DD), _f32).at[:NN].set(x)
    z1 = jnp.zeros((NP,), _f32)
    z2 = jnp.zeros((NP, HALF), _f32)
    w2p = jnp.zeros((DD, HALF), _f32).at[:, 0].set(W2[:, 0])
    b2v16 = jnp.broadcast_to(b2, (16,)).astype(_f32)

    d0, d1 = _deg_call(dst32, z1)
    h1, g0, g1, dinv = _mm1_call(xp, W1, d0.reshape(NP, 1), d1.reshape(NP, 1))
    bias_rz = jnp.concatenate([b_hh[:2 * DD], jnp.zeros((DD,), _f32)])
    s0, s1 = _mp_call(g0, g1, src_p.reshape(16, NCHK, CHK),
                      dst_p.reshape(16, NCHK, CHK), z2)
    gi = _gi_call(s0, s1, h1, dinv, b1.reshape(1, DD), W_ih.T,
                  (b_ih + bias_rz).reshape(1, 3 * DD))
    q2d = _gru_call(gi, W_hh.T.astype(jnp.bfloat16),
                    b_hh[2 * DD:].reshape(1, DD), w2p, dinv)
    q1d = q2d[:, 0]
    out = _fin_call(q1d, src16, dst16, dinv.reshape(NP), b2v16, z1)
    return out[:NN]


# speedup vs baseline: 1.1143x; 1.1143x over previous
import jax
import jax.numpy as jnp
from jax import lax
from jax.experimental import pallas as pl
from jax.experimental.pallas import tpu as pltpu
from jax.experimental.pallas import tpu_sc as plsc

NN = 10000
EE = 160000
DD = 256
NP = 10240
EP = 163840
RB = 1024
NBLK = NP // RB
PAD_IDX = NP - 1
HALF = 128
CHK = 128
NCHK = EP // 16 // CHK

_f32 = jnp.float32


def _deg_body(dst_hbm, z1_hbm, d0_hbm, d1_hbm, dst_v, ones_v, acc_sp):
    c = lax.axis_index("c")
    s = lax.axis_index("s")
    wid = s * 2 + c

    @pl.when(s == 0)
    def _():
        pltpu.sync_copy(z1_hbm, acc_sp)

    plsc.subcore_barrier()
    pltpu.sync_copy(dst_hbm.at[wid], dst_v)
    for k in range(8):
        ones_v[pl.ds(k * 16, 16)] = jnp.ones((16,), _f32)

    def body(j, carry):
        pltpu.sync_copy(ones_v, acc_sp.at[dst_v.at[j]], add=True)
        return carry

    lax.fori_loop(0, 40, body, 0)
    plsc.subcore_barrier()
    rows = pl.ds(s * (NP // 16), NP // 16)

    @pl.when(c == 0)
    def _():
        pltpu.sync_copy(acc_sp.at[rows], d0_hbm.at[rows])

    @pl.when(c == 1)
    def _():
        pltpu.sync_copy(acc_sp.at[rows], d1_hbm.at[rows])


def _deg_call(dst32, z1):
    f = pl.kernel(
        _deg_body,
        out_type=[jax.ShapeDtypeStruct((NP,), _f32),
                  jax.ShapeDtypeStruct((NP,), _f32)],
        mesh=plsc.VectorSubcoreMesh(core_axis_name="c", subcore_axis_name="s", num_cores=2, num_subcores=16),
        scratch_types=[pltpu.VMEM((40, 128), jnp.int32),
                       pltpu.VMEM((128,), _f32),
                       pltpu.VMEM_SHARED((NP,), _f32)],
    )
    return f(dst32, z1)


def _mm1_body(x_ref, w1_ref, d0_ref, d1_ref, h1_ref, g0_ref, g1_ref, dinv_ref):
    deg = d0_ref[...] + d1_ref[...] + 1.0
    dinv = lax.rsqrt(deg)
    h1 = jnp.dot(x_ref[...], w1_ref[...], preferred_element_type=_f32)
    g = dinv * h1
    h1_ref[...] = h1
    g0_ref[...] = g[:, :HALF]
    g1_ref[...] = g[:, HALF:]
    dinv_ref[...] = dinv


def _mm1_call(xp, w1, d0, d1):
    return pl.pallas_call(
        _mm1_body,
        grid=(NBLK,),
        in_specs=[
            pl.BlockSpec((RB, DD), lambda i: (i, 0)),
            pl.BlockSpec((DD, DD), lambda i: (0, 0)),
            pl.BlockSpec((RB, 1), lambda i: (i, 0)),
            pl.BlockSpec((RB, 1), lambda i: (i, 0)),
        ],
        out_specs=[
            pl.BlockSpec((RB, DD), lambda i: (i, 0)),
            pl.BlockSpec((RB, HALF), lambda i: (i, 0)),
            pl.BlockSpec((RB, HALF), lambda i: (i, 0)),
            pl.BlockSpec((RB, 1), lambda i: (i, 0)),
        ],
        out_shape=[
            jax.ShapeDtypeStruct((NP, DD), _f32),
            jax.ShapeDtypeStruct((NP, HALF), _f32),
            jax.ShapeDtypeStruct((NP, HALF), _f32),
            jax.ShapeDtypeStruct((NP, 1), _f32),
        ],
        compiler_params=pltpu.CompilerParams(
            dimension_semantics=("arbitrary",)),
    )(xp, w1, d0, d1)


def _mp_body(g0_hbm, g1_hbm, src_hbm, dst_hbm, z2_hbm, s0_hbm, s1_hbm,
             src_v, dst_v, rows_a, acc_sp, sem_a):
    c = lax.axis_index("c")
    s = lax.axis_index("s")

    @pl.when(s == 0)
    def _():
        pltpu.sync_copy(z2_hbm, acc_sp)

    plsc.subcore_barrier()
    pltpu.sync_copy(src_hbm.at[s], src_v)
    pltpu.sync_copy(dst_hbm.at[s], dst_v)

    def mk(g_hbm):
        def body(j, carry):
            pltpu.async_copy(g_hbm.at[src_v.at[j]], rows_a, sem_a).wait()
            pltpu.sync_copy(rows_a, acc_sp.at[dst_v.at[j]], add=True)
            return carry
        return body

    @pl.when(c == 0)
    def _():
        lax.fori_loop(0, NCHK, mk(g0_hbm), 0)

    @pl.when(c == 1)
    def _():
        lax.fori_loop(0, NCHK, mk(g1_hbm), 0)

    plsc.subcore_barrier()
    rows = pl.ds(s * (NP // 16), NP // 16)

    @pl.when(c == 0)
    def _():
        pltpu.sync_copy(acc_sp.at[rows], s0_hbm.at[rows])

    @pl.when(c == 1)
    def _():
        pltpu.sync_copy(acc_sp.at[rows], s1_hbm.at[rows])


def _mp_call(g0, g1, src16, dst16, z2):
    f = pl.kernel(
        _mp_body,
        out_type=[jax.ShapeDtypeStruct((NP, HALF), _f32),
                  jax.ShapeDtypeStruct((NP, HALF), _f32)],
        mesh=plsc.VectorSubcoreMesh(core_axis_name="c", subcore_axis_name="s", num_cores=2, num_subcores=16),
        scratch_types=[pltpu.VMEM((NCHK, CHK), jnp.int32),
                       pltpu.VMEM((NCHK, CHK), jnp.int32),
                       pltpu.VMEM((CHK, HALF), _f32),
                       pltpu.VMEM_SHARED((NP, HALF), _f32),
                       pltpu.SemaphoreType.DMA],
    )
    return f(g0, g1, src16, dst16, z2)


def _gi_body(s0_ref, s1_ref, h1_ref, dinv_ref, b1_ref, wih_ref, bih_ref,
             gi_ref):
    dinv = dinv_ref[...]
    scat = jnp.concatenate([s0_ref[...], s1_ref[...]], axis=1).astype(_f32)
    out1 = jnp.maximum(
        dinv * scat + (dinv * dinv) * h1_ref[...] + b1_ref[...], 0.0)
    gi_ref[...] = (jnp.dot(out1, wih_ref[...], preferred_element_type=_f32)
                   + bih_ref[...])


def _gi_call(s0, s1, h1, dinv, b1r, wihT, bihr):
    return pl.pallas_call(
        _gi_body,
        grid=(NBLK,),
        in_specs=[
            pl.BlockSpec((RB, HALF), lambda i: (i, 0)),
            pl.BlockSpec((RB, HALF), lambda i: (i, 0)),
            pl.BlockSpec((RB, DD), lambda i: (i, 0)),
            pl.BlockSpec((RB, 1), lambda i: (i, 0)),
            pl.BlockSpec((1, DD), lambda i: (0, 0)),
            pl.BlockSpec((DD, 3 * DD), lambda i: (0, 0)),
            pl.BlockSpec((1, 3 * DD), lambda i: (0, 0)),
        ],
        out_specs=pl.BlockSpec((RB, 3 * DD), lambda i: (i, 0)),
        out_shape=jax.ShapeDtypeStruct((NP, 3 * DD), _f32),
        compiler_params=pltpu.CompilerParams(
            dimension_semantics=("arbitrary",)),
    )(s0, s1, h1, dinv, b1r, wihT, bihr)


def _gru_body(gi_ref, whh_ref, bhn_ref, w2_ref, dinv_ref, q_ref,
              h_scr, ys_scr):
    i = pl.program_id(0)

    @pl.when(i == 0)
    def _():
        h_scr[...] = jnp.zeros((1, DD), _f32)

    whh = whh_ref[...]
    bhn = bhn_ref[...]

    def step(t, h):
        gi = gi_ref[pl.ds(t, 1), :]
        gh = jnp.dot(h.astype(jnp.bfloat16), whh, preferred_element_type=_f32)
        r = 0.5 * jnp.tanh(0.5 * (gi[:, :DD] + gh[:, :DD])) + 0.5
        z = 0.5 * jnp.tanh(0.5 * (gi[:, DD:2 * DD] + gh[:, DD:2 * DD])) + 0.5
        ng = jnp.tanh(gi[:, 2 * DD:] + r * (gh[:, 2 * DD:] + bhn))
        h = ng + z * (h - ng)
        ys_scr[pl.ds(t, 1), :] = h
        return h

    def step4(t4, h):
        t = 4 * t4
        return step(t + 3, step(t + 2, step(t + 1, step(t, h))))

    h = lax.fori_loop(0, RB // 4, step4, h_scr[...])
    h_scr[...] = h
    yb = jnp.dot(ys_scr[...], w2_ref[...], preferred_element_type=_f32)
    q_ref[...] = dinv_ref[...] * yb


def _gru_call(gi, whhT_bf, bhn, w2p, dinv):
    return pl.pallas_call(
        _gru_body,
        grid=(NBLK,),
        in_specs=[
            pl.BlockSpec((RB, 3 * DD), lambda i: (i, 0)),
            pl.BlockSpec((DD, 3 * DD), lambda i: (0, 0)),
            pl.BlockSpec((1, DD), lambda i: (0, 0)),
            pl.BlockSpec((DD, HALF), lambda i: (0, 0)),
            pl.BlockSpec((RB, 1), lambda i: (i, 0)),
        ],
        out_specs=pl.BlockSpec((RB, HALF), lambda i: (i, 0)),
        out_shape=jax.ShapeDtypeStruct((NP, HALF), _f32),
        scratch_shapes=[pltpu.VMEM((1, DD), _f32),
                        pltpu.VMEM((RB, DD), _f32)],
        compiler_params=pltpu.CompilerParams(
            dimension_semantics=("arbitrary",)),
    )(gi, whhT_bf, bhn, w2p, dinv)


def _fin_body(q_hbm, src_hbm, dst_hbm, dinv_hbm, b2_hbm, z1_hbm, out_hbm,
              src_v, dst_v, vals_v, accv, qv, dv, outv, b2v, acc_sp, sem):
    c = lax.axis_index("c")
    s = lax.axis_index("s")

    @pl.when(c == 0)
    def _():
        @pl.when(s == 0)
        def _():
            pltpu.sync_copy(z1_hbm, acc_sp)

        plsc.subcore_barrier()
        pltpu.sync_copy(src_hbm.at[s], src_v)
        pltpu.sync_copy(dst_hbm.at[s], dst_v)

        def body(j, carry):
            pltpu.async_copy(q_hbm.at[src_v.at[j]], vals_v, sem).wait()
            pltpu.sync_copy(vals_v, acc_sp.at[dst_v.at[j]], add=True)
            return carry

        lax.fori_loop(0, 80, body, 0)
        plsc.subcore_barrier()
        rows = pl.ds(s * (NP // 16), NP // 16)
        pltpu.sync_copy(acc_sp.at[rows], accv)
        pltpu.sync_copy(q_hbm.at[rows], qv)
        pltpu.sync_copy(dinv_hbm.at[rows], dv)
        pltpu.sync_copy(b2_hbm, b2v)
        b2x = b2v[...]
        for k in range(NP // 16 // 16):
            sl = pl.ds(k * 16, 16)
            outv[sl] = dv[sl] * (accv[sl] + qv[sl]) + b2x
        pltpu.sync_copy(outv, out_hbm.at[rows])


def _fin_call(q1d, src16, dst16, dinv1d, b2v16, z1):
    f = pl.kernel(
        _fin_body,
        out_type=jax.ShapeDtypeStruct((NP,), _f32),
        mesh=plsc.VectorSubcoreMesh(core_axis_name="c", subcore_axis_name="s", num_cores=2, num_subcores=16),
        scratch_types=[pltpu.VMEM((80, 128), jnp.int32),
                       pltpu.VMEM((80, 128), jnp.int32),
                       pltpu.VMEM((128,), _f32),
                       pltpu.VMEM((NP // 16,), _f32),
                       pltpu.VMEM((NP // 16,), _f32),
                       pltpu.VMEM((NP // 16,), _f32),
                       pltpu.VMEM((NP // 16,), _f32),
                       pltpu.VMEM((16,), _f32),
                       pltpu.VMEM_SHARED((NP,), _f32),
                       pltpu.SemaphoreType.DMA],
    )
    return f(q1d, src16, dst16, dinv1d, b2v16, z1)


def kernel(x, edge_index, W1, b1, W_ih, W_hh, b_ih, b_hh, W2, b2):
    src = edge_index[0].astype(jnp.int32)
    dst = edge_index[1].astype(jnp.int32)
    pad = jnp.full((EP - EE,), PAD_IDX, jnp.int32)
    src_p = jnp.concatenate([src, pad])
    dst_p = jnp.concatenate([dst, pad])
    src32 = src_p.reshape(32, 40, 128)
    dst32 = dst_p.reshape(32, 40, 128)
    src16 = src_p.reshape(16, 80, 128)
    dst16 = dst_p.reshape(16, 80, 128)

    xp = jnp.zeros((NP, DD), _f32).at[:NN].set(x)
    z1 = jnp.zeros((NP,), _f32)
    z2 = jnp.zeros((NP, HALF), _f32)
    w2p = jnp.zeros((DD, HALF), _f32).at[:, 0].set(W2[:, 0])
    b2v16 = jnp.broadcast_to(b2, (16,)).astype(_f32)

    d0, d1 = _deg_call(dst32, z1)
    h1, g0, g1, dinv = _mm1_call(xp, W1, d0.reshape(NP, 1), d1.reshape(NP, 1))
    bias_rz = jnp.concatenate([b_hh[:2 * DD], jnp.zeros((DD,), _f32)])
    s0, s1 = _mp_call(g0, g1, src_p.reshape(16, NCHK, CHK),
                      dst_p.reshape(16, NCHK, CHK), z2)
    gi = _gi_call(s0, s1, h1, dinv, b1.reshape(1, DD), W_ih.T,
                  (b_ih + bias_rz).reshape(1, 3 * DD))
    q2d = _gru_call(gi, W_hh.T.astype(jnp.bfloat16),
                    b_hh[2 * DD:].reshape(1, DD), w2p, dinv)
    q1d = q2d[:, 0]
    out = _fin_call(q1d, src16, dst16, dinv.reshape(NP), b2v16, z1)
    return out[:NN]


# speedup vs baseline: 1.1401x; 1.0231x over previous
import jax
import jax.numpy as jnp
from jax import lax
from jax.experimental import pallas as pl
from jax.experimental.pallas import tpu as pltpu
from jax.experimental.pallas import tpu_sc as plsc

NN = 10000
EE = 160000
DD = 256
NP = 10240
EP = 163840
RB = 1024
NBLK = NP // RB
PAD_IDX = NP - 1
HALF = 128
CHK = 128
NCHK = EP // 16 // CHK

_f32 = jnp.float32


def _deg_body(dst_hbm, z1_hbm, d0_hbm, d1_hbm, dst_v, ones_v, acc_sp):
    c = lax.axis_index("c")
    s = lax.axis_index("s")
    wid = s * 2 + c

    @pl.when(s == 0)
    def _():
        pltpu.sync_copy(z1_hbm, acc_sp)

    plsc.subcore_barrier()
    pltpu.sync_copy(dst_hbm.at[wid], dst_v)
    for k in range(8):
        ones_v[pl.ds(k * 16, 16)] = jnp.ones((16,), _f32)

    def body(j, carry):
        pltpu.sync_copy(ones_v, acc_sp.at[dst_v.at[j]], add=True)
        return carry

    lax.fori_loop(0, 40, body, 0)
    plsc.subcore_barrier()
    rows = pl.ds(s * (NP // 16), NP // 16)

    @pl.when(c == 0)
    def _():
        pltpu.sync_copy(acc_sp.at[rows], d0_hbm.at[rows])

    @pl.when(c == 1)
    def _():
        pltpu.sync_copy(acc_sp.at[rows], d1_hbm.at[rows])


def _deg_call(dst32, z1):
    f = pl.kernel(
        _deg_body,
        out_type=[jax.ShapeDtypeStruct((NP,), _f32),
                  jax.ShapeDtypeStruct((NP,), _f32)],
        mesh=plsc.VectorSubcoreMesh(core_axis_name="c", subcore_axis_name="s", num_cores=2, num_subcores=16),
        scratch_types=[pltpu.VMEM((40, 128), jnp.int32),
                       pltpu.VMEM((128,), _f32),
                       pltpu.VMEM_SHARED((NP,), _f32)],
    )
    return f(dst32, z1)


def _mm1_body(x_ref, w1_ref, d0_ref, d1_ref, h1_ref, g0_ref, g1_ref, dinv_ref):
    deg = d0_ref[...] + d1_ref[...] + 1.0
    dinv = lax.rsqrt(deg)
    h1 = jnp.dot(x_ref[...], w1_ref[...], preferred_element_type=_f32)
    g = dinv * h1
    h1_ref[...] = h1
    g0_ref[...] = g[:, :HALF]
    g1_ref[...] = g[:, HALF:]
    dinv_ref[...] = dinv


def _mm1_call(xp, w1, d0, d1):
    return pl.pallas_call(
        _mm1_body,
        grid=(NBLK,),
        in_specs=[
            pl.BlockSpec((RB, DD), lambda i: (i, 0)),
            pl.BlockSpec((DD, DD), lambda i: (0, 0)),
            pl.BlockSpec((RB, 1), lambda i: (i, 0)),
            pl.BlockSpec((RB, 1), lambda i: (i, 0)),
        ],
        out_specs=[
            pl.BlockSpec((RB, DD), lambda i: (i, 0)),
            pl.BlockSpec((RB, HALF), lambda i: (i, 0)),
            pl.BlockSpec((RB, HALF), lambda i: (i, 0)),
            pl.BlockSpec((RB, 1), lambda i: (i, 0)),
        ],
        out_shape=[
            jax.ShapeDtypeStruct((NP, DD), _f32),
            jax.ShapeDtypeStruct((NP, HALF), _f32),
            jax.ShapeDtypeStruct((NP, HALF), _f32),
            jax.ShapeDtypeStruct((NP, 1), _f32),
        ],
        compiler_params=pltpu.CompilerParams(
            dimension_semantics=("arbitrary",)),
    )(xp, w1, d0, d1)


def _mp_body(g0_hbm, g1_hbm, src_hbm, dst_hbm, z2_hbm, s0_hbm, s1_hbm,
             src_v, dst_v, rows_a, acc_sp, sem_a):
    c = lax.axis_index("c")
    s = lax.axis_index("s")

    @pl.when(s == 0)
    def _():
        pltpu.sync_copy(z2_hbm, acc_sp)

    plsc.subcore_barrier()
    pltpu.sync_copy(src_hbm.at[s], src_v)
    pltpu.sync_copy(dst_hbm.at[s], dst_v)

    def mk(g_hbm):
        def body(j, carry):
            pltpu.async_copy(g_hbm.at[src_v.at[j]], rows_a, sem_a).wait()
            pltpu.sync_copy(rows_a, acc_sp.at[dst_v.at[j]], add=True)
            return carry
        return body

    @pl.when(c == 0)
    def _():
        lax.fori_loop(0, NCHK, mk(g0_hbm), 0)

    @pl.when(c == 1)
    def _():
        lax.fori_loop(0, NCHK, mk(g1_hbm), 0)

    plsc.subcore_barrier()
    rows = pl.ds(s * (NP // 16), NP // 16)

    @pl.when(c == 0)
    def _():
        pltpu.sync_copy(acc_sp.at[rows], s0_hbm.at[rows])

    @pl.when(c == 1)
    def _():
        pltpu.sync_copy(acc_sp.at[rows], s1_hbm.at[rows])


def _mp_call(g0, g1, src16, dst16, z2):
    f = pl.kernel(
        _mp_body,
        out_type=[jax.ShapeDtypeStruct((NP, HALF), _f32),
                  jax.ShapeDtypeStruct((NP, HALF), _f32)],
        mesh=plsc.VectorSubcoreMesh(core_axis_name="c", subcore_axis_name="s", num_cores=2, num_subcores=16),
        scratch_types=[pltpu.VMEM((NCHK, CHK), jnp.int32),
                       pltpu.VMEM((NCHK, CHK), jnp.int32),
                       pltpu.VMEM((CHK, HALF), _f32),
                       pltpu.VMEM_SHARED((NP, HALF), _f32),
                       pltpu.SemaphoreType.DMA],
    )
    return f(g0, g1, src16, dst16, z2)


def _gi_body(s0_ref, s1_ref, h1_ref, dinv_ref, b1_ref, wih_ref, bih_ref,
             gi_ref):
    dinv = dinv_ref[...]
    scat = jnp.concatenate([s0_ref[...], s1_ref[...]], axis=1).astype(_f32)
    out1 = jnp.maximum(
        dinv * scat + (dinv * dinv) * h1_ref[...] + b1_ref[...], 0.0)
    gi_ref[...] = (jnp.dot(out1, wih_ref[...], preferred_element_type=_f32)
                   + bih_ref[...])


def _gi_call(s0, s1, h1, dinv, b1r, wihT, bihr):
    return pl.pallas_call(
        _gi_body,
        grid=(NBLK,),
        in_specs=[
            pl.BlockSpec((RB, HALF), lambda i: (i, 0)),
            pl.BlockSpec((RB, HALF), lambda i: (i, 0)),
            pl.BlockSpec((RB, DD), lambda i: (i, 0)),
            pl.BlockSpec((RB, 1), lambda i: (i, 0)),
            pl.BlockSpec((1, DD), lambda i: (0, 0)),
            pl.BlockSpec((DD, 3 * DD), lambda i: (0, 0)),
            pl.BlockSpec((1, 3 * DD), lambda i: (0, 0)),
        ],
        out_specs=pl.BlockSpec((RB, 3 * DD), lambda i: (i, 0)),
        out_shape=jax.ShapeDtypeStruct((NP, 3 * DD), _f32),
        compiler_params=pltpu.CompilerParams(
            dimension_semantics=("arbitrary",)),
    )(s0, s1, h1, dinv, b1r, wihT, bihr)


def _gru_body(gi_ref, whh_ref, bhn_ref, w2_ref, dinv_ref, q_ref,
              h_scr, ys_scr):
    i = pl.program_id(0)

    @pl.when(i == 0)
    def _():
        h_scr[...] = jnp.zeros((1, DD), _f32)

    whh = whh_ref[...]
    bhn = bhn_ref[...]

    def step(t, h):
        gi = gi_ref[pl.ds(t, 1), :]
        gh = jnp.dot(h.astype(jnp.bfloat16), whh, preferred_element_type=_f32)
        r = 0.5 * jnp.tanh(0.5 * (gi[:, :DD] + gh[:, :DD])) + 0.5
        z = 0.5 * jnp.tanh(0.5 * (gi[:, DD:2 * DD] + gh[:, DD:2 * DD])) + 0.5
        ng = jnp.tanh(gi[:, 2 * DD:] + r * (gh[:, 2 * DD:] + bhn))
        h = ng + z * (h - ng)
        ys_scr[pl.ds(t, 1), :] = h
        return h

    def step8(t8, h):
        t = 8 * t8
        for u in range(8):
            h = step(t + u, h)
        return h

    h = lax.fori_loop(0, RB // 8, step8, h_scr[...])
    h_scr[...] = h
    yb = jnp.dot(ys_scr[...], w2_ref[...], preferred_element_type=_f32)
    q_ref[...] = dinv_ref[...] * yb


def _gru_call(gi, whhT_bf, bhn, w2p, dinv):
    return pl.pallas_call(
        _gru_body,
        grid=(NBLK,),
        in_specs=[
            pl.BlockSpec((RB, 3 * DD), lambda i: (i, 0)),
            pl.BlockSpec((DD, 3 * DD), lambda i: (0, 0)),
            pl.BlockSpec((1, DD), lambda i: (0, 0)),
            pl.BlockSpec((DD, HALF), lambda i: (0, 0)),
            pl.BlockSpec((RB, 1), lambda i: (i, 0)),
        ],
        out_specs=pl.BlockSpec((RB, HALF), lambda i: (i, 0)),
        out_shape=jax.ShapeDtypeStruct((NP, HALF), _f32),
        scratch_shapes=[pltpu.VMEM((1, DD), _f32),
                        pltpu.VMEM((RB, DD), _f32)],
        compiler_params=pltpu.CompilerParams(
            dimension_semantics=("arbitrary",)),
    )(gi, whhT_bf, bhn, w2p, dinv)


def _fin_body(q_hbm, src_hbm, dst_hbm, dinv_hbm, b2_hbm, z1_hbm, out_hbm,
              src_v, dst_v, vals_v, accv, qv, dv, outv, b2v, acc_sp, sem):
    c = lax.axis_index("c")
    s = lax.axis_index("s")

    @pl.when(c == 0)
    def _():
        @pl.when(s == 0)
        def _():
            pltpu.sync_copy(z1_hbm, acc_sp)

        plsc.subcore_barrier()
        pltpu.sync_copy(src_hbm.at[s], src_v)
        pltpu.sync_copy(dst_hbm.at[s], dst_v)

        def body(j, carry):
            pltpu.async_copy(q_hbm.at[src_v.at[j]], vals_v, sem).wait()
            pltpu.sync_copy(vals_v, acc_sp.at[dst_v.at[j]], add=True)
            return carry

        lax.fori_loop(0, 80, body, 0)
        plsc.subcore_barrier()
        rows = pl.ds(s * (NP // 16), NP // 16)
        pltpu.sync_copy(acc_sp.at[rows], accv)
        pltpu.sync_copy(q_hbm.at[rows], qv)
        pltpu.sync_copy(dinv_hbm.at[rows], dv)
        pltpu.sync_copy(b2_hbm, b2v)
        b2x = b2v[...]
        for k in range(NP // 16 // 16):
            sl = pl.ds(k * 16, 16)
            outv[sl] = dv[sl] * (accv[sl] + qv[sl]) + b2x
        pltpu.sync_copy(outv, out_hbm.at[rows])


def _fin_call(q1d, src16, dst16, dinv1d, b2v16, z1):
    f = pl.kernel(
        _fin_body,
        out_type=jax.ShapeDtypeStruct((NP,), _f32),
        mesh=plsc.VectorSubcoreMesh(core_axis_name="c", subcore_axis_name="s", num_cores=2, num_subcores=16),
        scratch_types=[pltpu.VMEM((80, 128), jnp.int32),
                       pltpu.VMEM((80, 128), jnp.int32),
                       pltpu.VMEM((128,), _f32),
                       pltpu.VMEM((NP // 16,), _f32),
                       pltpu.VMEM((NP // 16,), _f32),
                       pltpu.VMEM((NP // 16,), _f32),
                       pltpu.VMEM((NP // 16,), _f32),
                       pltpu.VMEM((16,), _f32),
                       pltpu.VMEM_SHARED((NP,), _f32),
                       pltpu.SemaphoreType.DMA],
    )
    return f(q1d, src16, dst16, dinv1d, b2v16, z1)


def kernel(x, edge_index, W1, b1, W_ih, W_hh, b_ih, b_hh, W2, b2):
    src = edge_index[0].astype(jnp.int32)
    dst = edge_index[1].astype(jnp.int32)
    pad = jnp.full((EP - EE,), PAD_IDX, jnp.int32)
    src_p = jnp.concatenate([src, pad])
    dst_p = jnp.concatenate([dst, pad])
    src32 = src_p.reshape(32, 40, 128)
    dst32 = dst_p.reshape(32, 40, 128)
    src16 = src_p.reshape(16, 80, 128)
    dst16 = dst_p.reshape(16, 80, 128)

    xp = jnp.zeros((NP, DD), _f32).at[:NN].set(x)
    z1 = jnp.zeros((NP,), _f32)
    z2 = jnp.zeros((NP, HALF), _f32)
    w2p = jnp.zeros((DD, HALF), _f32).at[:, 0].set(W2[:, 0])
    b2v16 = jnp.broadcast_to(b2, (16,)).astype(_f32)

    d0, d1 = _deg_call(dst32, z1)
    h1, g0, g1, dinv = _mm1_call(xp, W1, d0.reshape(NP, 1), d1.reshape(NP, 1))
    bias_rz = jnp.concatenate([b_hh[:2 * DD], jnp.zeros((DD,), _f32)])
    s0, s1 = _mp_call(g0, g1, src_p.reshape(16, NCHK, CHK),
                      dst_p.reshape(16, NCHK, CHK), z2)
    gi = _gi_call(s0, s1, h1, dinv, b1.reshape(1, DD), W_ih.T,
                  (b_ih + bias_rz).reshape(1, 3 * DD))
    q2d = _gru_call(gi, W_hh.T.astype(jnp.bfloat16),
                    b_hh[2 * DD:].reshape(1, DD), w2p, dinv)
    q1d = q2d[:, 0]
    out = _fin_call(q1d, src16, dst16, dinv.reshape(NP), b2v16, z1)
    return out[:NN]


# speedup vs baseline: 1.1482x; 1.0071x over previous
import jax
import jax.numpy as jnp
from jax import lax
from jax.experimental import pallas as pl
from jax.experimental.pallas import tpu as pltpu
from jax.experimental.pallas import tpu_sc as plsc

NN = 10000
EE = 160000
DD = 256
NP = 10240
EP = 163840
RB = 1024
NBLK = NP // RB
PAD_IDX = NP - 1
HALF = 128
CHK = 128
NCHK = EP // 16 // CHK

_f32 = jnp.float32


def _deg_body(dst_hbm, z1_hbm, d0_hbm, d1_hbm, dst_v, ones_v, acc_sp):
    c = lax.axis_index("c")
    s = lax.axis_index("s")
    wid = s * 2 + c

    @pl.when(s == 0)
    def _():
        pltpu.sync_copy(z1_hbm, acc_sp)

    plsc.subcore_barrier()
    pltpu.sync_copy(dst_hbm.at[wid], dst_v)
    for k in range(8):
        ones_v[pl.ds(k * 16, 16)] = jnp.ones((16,), _f32)

    def body(j, carry):
        pltpu.sync_copy(ones_v, acc_sp.at[dst_v.at[j]], add=True)
        return carry

    lax.fori_loop(0, 40, body, 0)
    plsc.subcore_barrier()
    rows = pl.ds(s * (NP // 16), NP // 16)

    @pl.when(c == 0)
    def _():
        pltpu.sync_copy(acc_sp.at[rows], d0_hbm.at[rows])

    @pl.when(c == 1)
    def _():
        pltpu.sync_copy(acc_sp.at[rows], d1_hbm.at[rows])


def _deg_call(dst32, z1):
    f = pl.kernel(
        _deg_body,
        out_type=[jax.ShapeDtypeStruct((NP,), _f32),
                  jax.ShapeDtypeStruct((NP,), _f32)],
        mesh=plsc.VectorSubcoreMesh(core_axis_name="c", subcore_axis_name="s", num_cores=2, num_subcores=16),
        scratch_types=[pltpu.VMEM((40, 128), jnp.int32),
                       pltpu.VMEM((128,), _f32),
                       pltpu.VMEM_SHARED((NP,), _f32)],
    )
    return f(dst32, z1)


def _mm1_body(x_ref, w1_ref, d0_ref, d1_ref, h1_ref, g0_ref, g1_ref, dinv_ref):
    deg = d0_ref[...] + d1_ref[...] + 1.0
    dinv = lax.rsqrt(deg)
    h1 = jnp.dot(x_ref[...], w1_ref[...], preferred_element_type=_f32)
    g = dinv * h1
    h1_ref[...] = h1
    g0_ref[...] = g[:, :HALF]
    g1_ref[...] = g[:, HALF:]
    dinv_ref[...] = dinv


def _mm1_call(xp, w1, d0, d1):
    return pl.pallas_call(
        _mm1_body,
        grid=(NBLK,),
        in_specs=[
            pl.BlockSpec((RB, DD), lambda i: (i, 0)),
            pl.BlockSpec((DD, DD), lambda i: (0, 0)),
            pl.BlockSpec((RB, 1), lambda i: (i, 0)),
            pl.BlockSpec((RB, 1), lambda i: (i, 0)),
        ],
        out_specs=[
            pl.BlockSpec((RB, DD), lambda i: (i, 0)),
            pl.BlockSpec((RB, HALF), lambda i: (i, 0)),
            pl.BlockSpec((RB, HALF), lambda i: (i, 0)),
            pl.BlockSpec((RB, 1), lambda i: (i, 0)),
        ],
        out_shape=[
            jax.ShapeDtypeStruct((NP, DD), _f32),
            jax.ShapeDtypeStruct((NP, HALF), _f32),
            jax.ShapeDtypeStruct((NP, HALF), _f32),
            jax.ShapeDtypeStruct((NP, 1), _f32),
        ],
        compiler_params=pltpu.CompilerParams(
            dimension_semantics=("arbitrary",)),
    )(xp, w1, d0, d1)


def _mp_body(g0_hbm, g1_hbm, src_hbm, dst_hbm, z2_hbm, s0_hbm, s1_hbm,
             src_v, dst_v, rows_a, rows_b, acc_sp, sem_a, sem_b):
    c = lax.axis_index("c")
    s = lax.axis_index("s")

    @pl.when(s == 0)
    def _():
        pltpu.sync_copy(z2_hbm, acc_sp)

    plsc.subcore_barrier()
    pltpu.sync_copy(dst_hbm.at[s], dst_v)

    def half(g_hbm, hh):
        pltpu.sync_copy(src_hbm.at[s, pl.ds(hh * (NCHK // 2), NCHK // 2)],
                        src_v)

        def body(i, carry):
            j = 2 * i
            da = pltpu.async_copy(g_hbm.at[src_v.at[j]], rows_a, sem_a)
            db = pltpu.async_copy(g_hbm.at[src_v.at[j + 1]], rows_b, sem_b)
            da.wait()
            pltpu.sync_copy(rows_a,
                            acc_sp.at[dst_v.at[hh * (NCHK // 2) + j]],
                            add=True)
            db.wait()
            pltpu.sync_copy(rows_b,
                            acc_sp.at[dst_v.at[hh * (NCHK // 2) + j + 1]],
                            add=True)
            return carry

        lax.fori_loop(0, NCHK // 4, body, 0)

    @pl.when(c == 0)
    def _():
        half(g0_hbm, 0)
        half(g0_hbm, 1)

    @pl.when(c == 1)
    def _():
        half(g1_hbm, 0)
        half(g1_hbm, 1)

    plsc.subcore_barrier()
    rows = pl.ds(s * (NP // 16), NP // 16)

    @pl.when(c == 0)
    def _():
        pltpu.sync_copy(acc_sp.at[rows], s0_hbm.at[rows])

    @pl.when(c == 1)
    def _():
        pltpu.sync_copy(acc_sp.at[rows], s1_hbm.at[rows])


def _mp_call(g0, g1, src16, dst16, z2):
    f = pl.kernel(
        _mp_body,
        out_type=[jax.ShapeDtypeStruct((NP, HALF), _f32),
                  jax.ShapeDtypeStruct((NP, HALF), _f32)],
        mesh=plsc.VectorSubcoreMesh(core_axis_name="c", subcore_axis_name="s", num_cores=2, num_subcores=16),
        scratch_types=[pltpu.VMEM((NCHK // 2, CHK), jnp.int32),
                       pltpu.VMEM((NCHK, CHK), jnp.int32),
                       pltpu.VMEM((CHK, HALF), _f32),
                       pltpu.VMEM((CHK, HALF), _f32),
                       pltpu.VMEM_SHARED((NP, HALF), _f32),
                       pltpu.SemaphoreType.DMA,
                       pltpu.SemaphoreType.DMA],
    )
    return f(g0, g1, src16, dst16, z2)


def _gi_body(s0_ref, s1_ref, h1_ref, dinv_ref, b1_ref, wih_ref, bih_ref,
             gi_ref):
    dinv = dinv_ref[...]
    scat = jnp.concatenate([s0_ref[...], s1_ref[...]], axis=1).astype(_f32)
    out1 = jnp.maximum(
        dinv * scat + (dinv * dinv) * h1_ref[...] + b1_ref[...], 0.0)
    gi_ref[...] = (jnp.dot(out1, wih_ref[...], preferred_element_type=_f32)
                   + bih_ref[...])


def _gi_call(s0, s1, h1, dinv, b1r, wihT, bihr):
    return pl.pallas_call(
        _gi_body,
        grid=(NBLK,),
        in_specs=[
            pl.BlockSpec((RB, HALF), lambda i: (i, 0)),
            pl.BlockSpec((RB, HALF), lambda i: (i, 0)),
            pl.BlockSpec((RB, DD), lambda i: (i, 0)),
            pl.BlockSpec((RB, 1), lambda i: (i, 0)),
            pl.BlockSpec((1, DD), lambda i: (0, 0)),
            pl.BlockSpec((DD, 3 * DD), lambda i: (0, 0)),
            pl.BlockSpec((1, 3 * DD), lambda i: (0, 0)),
        ],
        out_specs=pl.BlockSpec((RB, 3 * DD), lambda i: (i, 0)),
        out_shape=jax.ShapeDtypeStruct((NP, 3 * DD), _f32),
        compiler_params=pltpu.CompilerParams(
            dimension_semantics=("arbitrary",)),
    )(s0, s1, h1, dinv, b1r, wihT, bihr)


def _gru_body(gi_ref, whh_ref, bhn_ref, w2_ref, dinv_ref, q_ref,
              h_scr, ys_scr):
    i = pl.program_id(0)

    @pl.when(i == 0)
    def _():
        h_scr[...] = jnp.zeros((1, DD), _f32)

    whh = whh_ref[...]
    bhn = bhn_ref[...]

    def step(t, h):
        gi = gi_ref[pl.ds(t, 1), :]
        gh = jnp.dot(h.astype(jnp.bfloat16), whh, preferred_element_type=_f32)
        r = 0.5 * jnp.tanh(0.5 * (gi[:, :DD] + gh[:, :DD])) + 0.5
        z = 0.5 * jnp.tanh(0.5 * (gi[:, DD:2 * DD] + gh[:, DD:2 * DD])) + 0.5
        ng = jnp.tanh(gi[:, 2 * DD:] + r * (gh[:, 2 * DD:] + bhn))
        h = ng + z * (h - ng)
        ys_scr[pl.ds(t, 1), :] = h
        return h

    def step8(t8, h):
        t = 8 * t8
        for u in range(8):
            h = step(t + u, h)
        return h

    h = lax.fori_loop(0, RB // 8, step8, h_scr[...])
    h_scr[...] = h
    yb = jnp.dot(ys_scr[...], w2_ref[...], preferred_element_type=_f32)
    q_ref[...] = dinv_ref[...] * yb


def _gru_call(gi, whhT_bf, bhn, w2p, dinv):
    return pl.pallas_call(
        _gru_body,
        grid=(NBLK,),
        in_specs=[
            pl.BlockSpec((RB, 3 * DD), lambda i: (i, 0)),
            pl.BlockSpec((DD, 3 * DD), lambda i: (0, 0)),
            pl.BlockSpec((1, DD), lambda i: (0, 0)),
            pl.BlockSpec((DD, HALF), lambda i: (0, 0)),
            pl.BlockSpec((RB, 1), lambda i: (i, 0)),
        ],
        out_specs=pl.BlockSpec((RB, HALF), lambda i: (i, 0)),
        out_shape=jax.ShapeDtypeStruct((NP, HALF), _f32),
        scratch_shapes=[pltpu.VMEM((1, DD), _f32),
                        pltpu.VMEM((RB, DD), _f32)],
        compiler_params=pltpu.CompilerParams(
            dimension_semantics=("arbitrary",)),
    )(gi, whhT_bf, bhn, w2p, dinv)


def _fin_body(q_hbm, src_hbm, dst_hbm, dinv_hbm, b2_hbm, z1_hbm, out_hbm,
              src_v, dst_v, vals_v, accv, qv, dv, outv, b2v, acc_sp, sem):
    c = lax.axis_index("c")
    s = lax.axis_index("s")

    @pl.when(c == 0)
    def _():
        @pl.when(s == 0)
        def _():
            pltpu.sync_copy(z1_hbm, acc_sp)

        plsc.subcore_barrier()
        pltpu.sync_copy(src_hbm.at[s], src_v)
        pltpu.sync_copy(dst_hbm.at[s], dst_v)

        def body(j, carry):
            pltpu.async_copy(q_hbm.at[src_v.at[j]], vals_v, sem).wait()
            pltpu.sync_copy(vals_v, acc_sp.at[dst_v.at[j]], add=True)
            return carry

        lax.fori_loop(0, 80, body, 0)
        plsc.subcore_barrier()
        rows = pl.ds(s * (NP // 16), NP // 16)
        pltpu.sync_copy(acc_sp.at[rows], accv)
        pltpu.sync_copy(q_hbm.at[rows], qv)
        pltpu.sync_copy(dinv_hbm.at[rows], dv)
        pltpu.sync_copy(b2_hbm, b2v)
        b2x = b2v[...]
        for k in range(NP // 16 // 16):
            sl = pl.ds(k * 16, 16)
            outv[sl] = dv[sl] * (accv[sl] + qv[sl]) + b2x
        pltpu.sync_copy(outv, out_hbm.at[rows])


def _fin_call(q1d, src16, dst16, dinv1d, b2v16, z1):
    f = pl.kernel(
        _fin_body,
        out_type=jax.ShapeDtypeStruct((NP,), _f32),
        mesh=plsc.VectorSubcoreMesh(core_axis_name="c", subcore_axis_name="s", num_cores=2, num_subcores=16),
        scratch_types=[pltpu.VMEM((80, 128), jnp.int32),
                       pltpu.VMEM((80, 128), jnp.int32),
                       pltpu.VMEM((128,), _f32),
                       pltpu.VMEM((NP // 16,), _f32),
                       pltpu.VMEM((NP // 16,), _f32),
                       pltpu.VMEM((NP // 16,), _f32),
                       pltpu.VMEM((NP // 16,), _f32),
                       pltpu.VMEM((16,), _f32),
                       pltpu.VMEM_SHARED((NP,), _f32),
                       pltpu.SemaphoreType.DMA],
    )
    return f(q1d, src16, dst16, dinv1d, b2v16, z1)


def kernel(x, edge_index, W1, b1, W_ih, W_hh, b_ih, b_hh, W2, b2):
    src = edge_index[0].astype(jnp.int32)
    dst = edge_index[1].astype(jnp.int32)
    pad = jnp.full((EP - EE,), PAD_IDX, jnp.int32)
    src_p = jnp.concatenate([src, pad])
    dst_p = jnp.concatenate([dst, pad])
    src32 = src_p.reshape(32, 40, 128)
    dst32 = dst_p.reshape(32, 40, 128)
    src16 = src_p.reshape(16, 80, 128)
    dst16 = dst_p.reshape(16, 80, 128)

    xp = jnp.zeros((NP, DD), _f32).at[:NN].set(x)
    z1 = jnp.zeros((NP,), _f32)
    z2 = jnp.zeros((NP, HALF), _f32)
    w2p = jnp.zeros((DD, HALF), _f32).at[:, 0].set(W2[:, 0])
    b2v16 = jnp.broadcast_to(b2, (16,)).astype(_f32)

    d0, d1 = _deg_call(dst32, z1)
    h1, g0, g1, dinv = _mm1_call(xp, W1, d0.reshape(NP, 1), d1.reshape(NP, 1))
    bias_rz = jnp.concatenate([b_hh[:2 * DD], jnp.zeros((DD,), _f32)])
    s0, s1 = _mp_call(g0, g1, src_p.reshape(16, NCHK, CHK),
                      dst_p.reshape(16, NCHK, CHK), z2)
    gi = _gi_call(s0, s1, h1, dinv, b1.reshape(1, DD), W_ih.T,
                  (b_ih + bias_rz).reshape(1, 3 * DD))
    q2d = _gru_call(gi, W_hh.T.astype(jnp.bfloat16),
                    b_hh[2 * DD:].reshape(1, DD), w2p, dinv)
    q1d = q2d[:, 0]
    out = _fin_call(q1d, src16, dst16, dinv.reshape(NP), b2v16, z1)
    return out[:NN]


# speedup vs baseline: 1.1634x; 1.0133x over previous
import jax
import jax.numpy as jnp
from jax import lax
from jax.experimental import pallas as pl
from jax.experimental.pallas import tpu as pltpu
from jax.experimental.pallas import tpu_sc as plsc

NN = 10000
EE = 160000
DD = 256
NP = 10240
EP = 163840
RB = 1024
NBLK = NP // RB
PAD_IDX = NP - 1
HALF = 128
CHK = 128
NCHK = EP // 16 // CHK

_f32 = jnp.float32


def _deg_body(dst_hbm, z1_hbm, d0_hbm, d1_hbm, dst_v, ones_v, acc_sp):
    c = lax.axis_index("c")
    s = lax.axis_index("s")
    wid = s * 2 + c

    @pl.when(s == 0)
    def _():
        pltpu.sync_copy(z1_hbm, acc_sp)

    plsc.subcore_barrier()
    pltpu.sync_copy(dst_hbm.at[wid], dst_v)
    for k in range(8):
        ones_v[pl.ds(k * 16, 16)] = jnp.ones((16,), _f32)

    def body(j, carry):
        pltpu.sync_copy(ones_v, acc_sp.at[dst_v.at[j]], add=True)
        return carry

    lax.fori_loop(0, 40, body, 0)
    plsc.subcore_barrier()
    rows = pl.ds(s * (NP // 16), NP // 16)

    @pl.when(c == 0)
    def _():
        pltpu.sync_copy(acc_sp.at[rows], d0_hbm.at[rows])

    @pl.when(c == 1)
    def _():
        pltpu.sync_copy(acc_sp.at[rows], d1_hbm.at[rows])


def _deg_call(dst32, z1):
    f = pl.kernel(
        _deg_body,
        out_type=[jax.ShapeDtypeStruct((NP,), _f32),
                  jax.ShapeDtypeStruct((NP,), _f32)],
        mesh=plsc.VectorSubcoreMesh(core_axis_name="c", subcore_axis_name="s", num_cores=2, num_subcores=16),
        scratch_types=[pltpu.VMEM((40, 128), jnp.int32),
                       pltpu.VMEM((128,), _f32),
                       pltpu.VMEM_SHARED((NP,), _f32)],
    )
    return f(dst32, z1)


def _mm1_body(x_ref, w1_ref, d0_ref, d1_ref, h1_ref, g0_ref, g1_ref, dinv_ref):
    deg = d0_ref[...] + d1_ref[...] + 1.0
    dinv = lax.rsqrt(deg)
    h1 = jnp.dot(x_ref[...], w1_ref[...], preferred_element_type=_f32)
    g = dinv * h1
    h1_ref[...] = h1
    g0_ref[...] = g[:, :HALF]
    g1_ref[...] = g[:, HALF:]
    dinv_ref[...] = dinv


def _mm1_call(xp, w1, d0, d1):
    return pl.pallas_call(
        _mm1_body,
        grid=(NBLK,),
        in_specs=[
            pl.BlockSpec((RB, DD), lambda i: (i, 0)),
            pl.BlockSpec((DD, DD), lambda i: (0, 0)),
            pl.BlockSpec((RB, 1), lambda i: (i, 0)),
            pl.BlockSpec((RB, 1), lambda i: (i, 0)),
        ],
        out_specs=[
            pl.BlockSpec((RB, DD), lambda i: (i, 0)),
            pl.BlockSpec((RB, HALF), lambda i: (i, 0)),
            pl.BlockSpec((RB, HALF), lambda i: (i, 0)),
            pl.BlockSpec((RB, 1), lambda i: (i, 0)),
        ],
        out_shape=[
            jax.ShapeDtypeStruct((NP, DD), _f32),
            jax.ShapeDtypeStruct((NP, HALF), _f32),
            jax.ShapeDtypeStruct((NP, HALF), _f32),
            jax.ShapeDtypeStruct((NP, 1), _f32),
        ],
        compiler_params=pltpu.CompilerParams(
            dimension_semantics=("arbitrary",)),
    )(xp, w1, d0, d1)


def _mp_body(g0_hbm, g1_hbm, src_hbm, dst_hbm, z2_hbm, s0_hbm, s1_hbm,
             src_v, dst_v, rows_a, rows_b, acc_sp, sem_a, sem_b):
    c = lax.axis_index("c")
    s = lax.axis_index("s")

    @pl.when(s == 0)
    def _():
        pltpu.sync_copy(z2_hbm, acc_sp)

    plsc.subcore_barrier()
    pltpu.sync_copy(dst_hbm.at[s], dst_v)

    def half(g_hbm, hh):
        pltpu.sync_copy(src_hbm.at[s, pl.ds(hh * (NCHK // 2), NCHK // 2)],
                        src_v)

        def body(i, carry):
            j = 2 * i
            da = pltpu.async_copy(g_hbm.at[src_v.at[j]], rows_a, sem_a)
            db = pltpu.async_copy(g_hbm.at[src_v.at[j + 1]], rows_b, sem_b)
            da.wait()
            pltpu.sync_copy(rows_a,
                            acc_sp.at[dst_v.at[hh * (NCHK // 2) + j]],
                            add=True)
            db.wait()
            pltpu.sync_copy(rows_b,
                            acc_sp.at[dst_v.at[hh * (NCHK // 2) + j + 1]],
                            add=True)
            return carry

        lax.fori_loop(0, NCHK // 4, body, 0)

    @pl.when(c == 0)
    def _():
        half(g0_hbm, 0)
        half(g0_hbm, 1)

    @pl.when(c == 1)
    def _():
        half(g1_hbm, 0)
        half(g1_hbm, 1)

    plsc.subcore_barrier()
    rows = pl.ds(s * (NP // 16), NP // 16)

    @pl.when(c == 0)
    def _():
        pltpu.sync_copy(acc_sp.at[rows], s0_hbm.at[rows])

    @pl.when(c == 1)
    def _():
        pltpu.sync_copy(acc_sp.at[rows], s1_hbm.at[rows])


def _mp_call(g0, g1, src16, dst16, z2):
    f = pl.kernel(
        _mp_body,
        out_type=[jax.ShapeDtypeStruct((NP, HALF), _f32),
                  jax.ShapeDtypeStruct((NP, HALF), _f32)],
        mesh=plsc.VectorSubcoreMesh(core_axis_name="c", subcore_axis_name="s", num_cores=2, num_subcores=16),
        scratch_types=[pltpu.VMEM((NCHK // 2, CHK), jnp.int32),
                       pltpu.VMEM((NCHK, CHK), jnp.int32),
                       pltpu.VMEM((CHK, HALF), _f32),
                       pltpu.VMEM((CHK, HALF), _f32),
                       pltpu.VMEM_SHARED((NP, HALF), _f32),
                       pltpu.SemaphoreType.DMA,
                       pltpu.SemaphoreType.DMA],
    )
    return f(g0, g1, src16, dst16, z2)


def _gi_body(s0_ref, s1_ref, h1_ref, dinv_ref, b1_ref, wih_ref, bih_ref,
             gi_ref):
    dinv = dinv_ref[...]
    scat = jnp.concatenate([s0_ref[...], s1_ref[...]], axis=1).astype(_f32)
    out1 = jnp.maximum(
        dinv * scat + (dinv * dinv) * h1_ref[...] + b1_ref[...], 0.0)
    gi_ref[...] = (jnp.dot(out1, wih_ref[...], preferred_element_type=_f32)
                   + bih_ref[...])


def _gi_call(s0, s1, h1, dinv, b1r, wihT, bihr):
    return pl.pallas_call(
        _gi_body,
        grid=(NBLK,),
        in_specs=[
            pl.BlockSpec((RB, HALF), lambda i: (i, 0)),
            pl.BlockSpec((RB, HALF), lambda i: (i, 0)),
            pl.BlockSpec((RB, DD), lambda i: (i, 0)),
            pl.BlockSpec((RB, 1), lambda i: (i, 0)),
            pl.BlockSpec((1, DD), lambda i: (0, 0)),
            pl.BlockSpec((DD, 3 * DD), lambda i: (0, 0)),
            pl.BlockSpec((1, 3 * DD), lambda i: (0, 0)),
        ],
        out_specs=pl.BlockSpec((RB, 3 * DD), lambda i: (i, 0)),
        out_shape=jax.ShapeDtypeStruct((NP, 3 * DD), _f32),
        compiler_params=pltpu.CompilerParams(
            dimension_semantics=("arbitrary",)),
    )(s0, s1, h1, dinv, b1r, wihT, bihr)


def _gru_body(gi_ref, whh_ref, bhn_ref, w2_ref, dinv_ref, q_ref,
              h_scr, ys_scr):
    i = pl.program_id(0)

    @pl.when(i == 0)
    def _():
        h_scr[...] = jnp.zeros((1, DD), _f32)

    whh = whh_ref[...]
    bhn = bhn_ref[...]

    def step(t, h):
        gi = gi_ref[pl.ds(t, 1), :]
        gh = jnp.dot(h.astype(jnp.bfloat16), whh, preferred_element_type=_f32)
        r = 0.5 * jnp.tanh(0.5 * (gi[:, :DD] + gh[:, :DD])) + 0.5
        z = 0.5 * jnp.tanh(0.5 * (gi[:, DD:2 * DD] + gh[:, DD:2 * DD])) + 0.5
        ng = jnp.tanh(gi[:, 2 * DD:] + r * (gh[:, 2 * DD:] + bhn))
        h = ng + z * (h - ng)
        ys_scr[pl.ds(t, 1), :] = h
        return h

    def step16(t16, h):
        t = 16 * t16
        for u in range(16):
            h = step(t + u, h)
        return h

    h = lax.fori_loop(0, RB // 16, step16, h_scr[...])
    h_scr[...] = h
    yb = jnp.dot(ys_scr[...], w2_ref[...], preferred_element_type=_f32)
    q_ref[...] = dinv_ref[...] * yb


def _gru_call(gi, whhT_bf, bhn, w2p, dinv):
    return pl.pallas_call(
        _gru_body,
        grid=(NBLK,),
        in_specs=[
            pl.BlockSpec((RB, 3 * DD), lambda i: (i, 0)),
            pl.BlockSpec((DD, 3 * DD), lambda i: (0, 0)),
            pl.BlockSpec((1, DD), lambda i: (0, 0)),
            pl.BlockSpec((DD, HALF), lambda i: (0, 0)),
            pl.BlockSpec((RB, 1), lambda i: (i, 0)),
        ],
        out_specs=pl.BlockSpec((RB, HALF), lambda i: (i, 0)),
        out_shape=jax.ShapeDtypeStruct((NP, HALF), _f32),
        scratch_shapes=[pltpu.VMEM((1, DD), _f32),
                        pltpu.VMEM((RB, DD), _f32)],
        compiler_params=pltpu.CompilerParams(
            dimension_semantics=("arbitrary",)),
    )(gi, whhT_bf, bhn, w2p, dinv)


def _fin_body(q_hbm, src_hbm, dst_hbm, dinv_hbm, b2_hbm, z1_hbm, out_hbm,
              src_v, dst_v, vals_v, accv, qv, dv, outv, b2v, acc_sp, sem):
    c = lax.axis_index("c")
    s = lax.axis_index("s")

    @pl.when(c == 0)
    def _():
        @pl.when(s == 0)
        def _():
            pltpu.sync_copy(z1_hbm, acc_sp)

        plsc.subcore_barrier()
        pltpu.sync_copy(src_hbm.at[s], src_v)
        pltpu.sync_copy(dst_hbm.at[s], dst_v)

        def body(j, carry):
            pltpu.async_copy(q_hbm.at[src_v.at[j]], vals_v, sem).wait()
            pltpu.sync_copy(vals_v, acc_sp.at[dst_v.at[j]], add=True)
            return carry

        lax.fori_loop(0, 80, body, 0)
        plsc.subcore_barrier()
        rows = pl.ds(s * (NP // 16), NP // 16)
        pltpu.sync_copy(acc_sp.at[rows], accv)
        pltpu.sync_copy(q_hbm.at[rows], qv)
        pltpu.sync_copy(dinv_hbm.at[rows], dv)
        pltpu.sync_copy(b2_hbm, b2v)
        b2x = b2v[...]
        for k in range(NP // 16 // 16):
            sl = pl.ds(k * 16, 16)
            outv[sl] = dv[sl] * (accv[sl] + qv[sl]) + b2x
        pltpu.sync_copy(outv, out_hbm.at[rows])


def _fin_call(q1d, src16, dst16, dinv1d, b2v16, z1):
    f = pl.kernel(
        _fin_body,
        out_type=jax.ShapeDtypeStruct((NP,), _f32),
        mesh=plsc.VectorSubcoreMesh(core_axis_name="c", subcore_axis_name="s", num_cores=2, num_subcores=16),
        scratch_types=[pltpu.VMEM((80, 128), jnp.int32),
                       pltpu.VMEM((80, 128), jnp.int32),
                       pltpu.VMEM((128,), _f32),
                       pltpu.VMEM((NP // 16,), _f32),
                       pltpu.VMEM((NP // 16,), _f32),
                       pltpu.VMEM((NP // 16,), _f32),
                       pltpu.VMEM((NP // 16,), _f32),
                       pltpu.VMEM((16,), _f32),
                       pltpu.VMEM_SHARED((NP,), _f32),
                       pltpu.SemaphoreType.DMA],
    )
    return f(q1d, src16, dst16, dinv1d, b2v16, z1)


def kernel(x, edge_index, W1, b1, W_ih, W_hh, b_ih, b_hh, W2, b2):
    src = edge_index[0].astype(jnp.int32)
    dst = edge_index[1].astype(jnp.int32)
    pad = jnp.full((EP - EE,), PAD_IDX, jnp.int32)
    src_p = jnp.concatenate([src, pad])
    dst_p = jnp.concatenate([dst, pad])
    src32 = src_p.reshape(32, 40, 128)
    dst32 = dst_p.reshape(32, 40, 128)
    src16 = src_p.reshape(16, 80, 128)
    dst16 = dst_p.reshape(16, 80, 128)

    xp = jnp.zeros((NP, DD), _f32).at[:NN].set(x)
    z1 = jnp.zeros((NP,), _f32)
    z2 = jnp.zeros((NP, HALF), _f32)
    w2p = jnp.zeros((DD, HALF), _f32).at[:, 0].set(W2[:, 0])
    b2v16 = jnp.broadcast_to(b2, (16,)).astype(_f32)

    d0, d1 = _deg_call(dst32, z1)
    h1, g0, g1, dinv = _mm1_call(xp, W1, d0.reshape(NP, 1), d1.reshape(NP, 1))
    bias_rz = jnp.concatenate([b_hh[:2 * DD], jnp.zeros((DD,), _f32)])
    s0, s1 = _mp_call(g0, g1, src_p.reshape(16, NCHK, CHK),
                      dst_p.reshape(16, NCHK, CHK), z2)
    gi = _gi_call(s0, s1, h1, dinv, b1.reshape(1, DD), W_ih.T,
                  (b_ih + bias_rz).reshape(1, 3 * DD))
    q2d = _gru_call(gi, W_hh.T.astype(jnp.bfloat16),
                    b_hh[2 * DD:].reshape(1, DD), w2p, dinv)
    q1d = q2d[:, 0]
    out = _fin_call(q1d, src16, dst16, dinv.reshape(NP), b2v16, z1)
    return out[:NN]


# speedup vs baseline: 1.1694x; 1.0052x over previous
import jax
import jax.numpy as jnp
from jax import lax
from jax.experimental import pallas as pl
from jax.experimental.pallas import tpu as pltpu
from jax.experimental.pallas import tpu_sc as plsc

NN = 10000
EE = 160000
DD = 256
NP = 10240
EP = 163840
RB = 1024
NBLK = NP // RB
PAD_IDX = NP - 1
HALF = 128
CHK = 128
NCHK = EP // 16 // CHK

_f32 = jnp.float32


def _deg_body(dst_hbm, z1_hbm, d0_hbm, d1_hbm, dst_v, ones_v, acc_sp):
    c = lax.axis_index("c")
    s = lax.axis_index("s")
    wid = s * 2 + c

    @pl.when(s == 0)
    def _():
        pltpu.sync_copy(z1_hbm, acc_sp)

    plsc.subcore_barrier()
    pltpu.sync_copy(dst_hbm.at[wid], dst_v)
    for k in range(8):
        ones_v[pl.ds(k * 16, 16)] = jnp.ones((16,), _f32)

    def body(j, carry):
        pltpu.sync_copy(ones_v, acc_sp.at[dst_v.at[j]], add=True)
        return carry

    lax.fori_loop(0, 40, body, 0)
    plsc.subcore_barrier()
    rows = pl.ds(s * (NP // 16), NP // 16)

    @pl.when(c == 0)
    def _():
        pltpu.sync_copy(acc_sp.at[rows], d0_hbm.at[rows])

    @pl.when(c == 1)
    def _():
        pltpu.sync_copy(acc_sp.at[rows], d1_hbm.at[rows])


def _deg_call(dst32, z1):
    f = pl.kernel(
        _deg_body,
        out_type=[jax.ShapeDtypeStruct((NP,), _f32),
                  jax.ShapeDtypeStruct((NP,), _f32)],
        mesh=plsc.VectorSubcoreMesh(core_axis_name="c", subcore_axis_name="s", num_cores=2, num_subcores=16),
        scratch_types=[pltpu.VMEM((40, 128), jnp.int32),
                       pltpu.VMEM((128,), _f32),
                       pltpu.VMEM_SHARED((NP,), _f32)],
    )
    return f(dst32, z1)


def _mm1_body(x_ref, w1_ref, d0_ref, d1_ref, h1_ref, g0_ref, g1_ref, dinv_ref):
    deg = d0_ref[...] + d1_ref[...] + 1.0
    dinv = lax.rsqrt(deg)
    h1 = jnp.dot(x_ref[...], w1_ref[...], preferred_element_type=_f32)
    g = dinv * h1
    h1_ref[...] = h1
    g0_ref[...] = g[:, :HALF]
    g1_ref[...] = g[:, HALF:]
    dinv_ref[...] = dinv


def _mm1_call(xp, w1, d0, d1):
    return pl.pallas_call(
        _mm1_body,
        grid=(NBLK,),
        in_specs=[
            pl.BlockSpec((RB, DD), lambda i: (i, 0)),
            pl.BlockSpec((DD, DD), lambda i: (0, 0)),
            pl.BlockSpec((RB, 1), lambda i: (i, 0)),
            pl.BlockSpec((RB, 1), lambda i: (i, 0)),
        ],
        out_specs=[
            pl.BlockSpec((RB, DD), lambda i: (i, 0)),
            pl.BlockSpec((RB, HALF), lambda i: (i, 0)),
            pl.BlockSpec((RB, HALF), lambda i: (i, 0)),
            pl.BlockSpec((RB, 1), lambda i: (i, 0)),
        ],
        out_shape=[
            jax.ShapeDtypeStruct((NP, DD), _f32),
            jax.ShapeDtypeStruct((NP, HALF), _f32),
            jax.ShapeDtypeStruct((NP, HALF), _f32),
            jax.ShapeDtypeStruct((NP, 1), _f32),
        ],
        compiler_params=pltpu.CompilerParams(
            dimension_semantics=("arbitrary",)),
    )(xp, w1, d0, d1)


def _mp_body(g0_hbm, g1_hbm, src_hbm, dst_hbm, z2_hbm, s0_hbm, s1_hbm,
             src_v, dst_v, rows_a, rows_b, acc_sp, sem_a, sem_b):
    c = lax.axis_index("c")
    s = lax.axis_index("s")

    @pl.when(s == 0)
    def _():
        pltpu.sync_copy(z2_hbm, acc_sp)

    plsc.subcore_barrier()
    pltpu.sync_copy(dst_hbm.at[s], dst_v)

    def half(g_hbm, hh):
        pltpu.sync_copy(src_hbm.at[s, pl.ds(hh * (NCHK // 2), NCHK // 2)],
                        src_v)

        def body(i, carry):
            j = 2 * i
            da = pltpu.async_copy(g_hbm.at[src_v.at[j]], rows_a, sem_a)
            db = pltpu.async_copy(g_hbm.at[src_v.at[j + 1]], rows_b, sem_b)
            da.wait()
            pltpu.sync_copy(rows_a,
                            acc_sp.at[dst_v.at[hh * (NCHK // 2) + j]],
                            add=True)
            db.wait()
            pltpu.sync_copy(rows_b,
                            acc_sp.at[dst_v.at[hh * (NCHK // 2) + j + 1]],
                            add=True)
            return carry

        lax.fori_loop(0, NCHK // 4, body, 0)

    @pl.when(c == 0)
    def _():
        half(g0_hbm, 0)
        half(g0_hbm, 1)

    @pl.when(c == 1)
    def _():
        half(g1_hbm, 0)
        half(g1_hbm, 1)

    plsc.subcore_barrier()
    rows = pl.ds(s * (NP // 16), NP // 16)

    @pl.when(c == 0)
    def _():
        pltpu.sync_copy(acc_sp.at[rows], s0_hbm.at[rows])

    @pl.when(c == 1)
    def _():
        pltpu.sync_copy(acc_sp.at[rows], s1_hbm.at[rows])


def _mp_call(g0, g1, src16, dst16, z2):
    f = pl.kernel(
        _mp_body,
        out_type=[jax.ShapeDtypeStruct((NP, HALF), _f32),
                  jax.ShapeDtypeStruct((NP, HALF), _f32)],
        mesh=plsc.VectorSubcoreMesh(core_axis_name="c", subcore_axis_name="s", num_cores=2, num_subcores=16),
        scratch_types=[pltpu.VMEM((NCHK // 2, CHK), jnp.int32),
                       pltpu.VMEM((NCHK, CHK), jnp.int32),
                       pltpu.VMEM((CHK, HALF), _f32),
                       pltpu.VMEM((CHK, HALF), _f32),
                       pltpu.VMEM_SHARED((NP, HALF), _f32),
                       pltpu.SemaphoreType.DMA,
                       pltpu.SemaphoreType.DMA],
    )
    return f(g0, g1, src16, dst16, z2)


def _gi_body(s0_ref, s1_ref, h1_ref, dinv_ref, b1_ref, wih_ref, bih_ref,
             gi_ref):
    dinv = dinv_ref[...]
    scat = jnp.concatenate([s0_ref[...], s1_ref[...]], axis=1).astype(_f32)
    out1 = jnp.maximum(
        dinv * scat + (dinv * dinv) * h1_ref[...] + b1_ref[...], 0.0)
    gi_ref[...] = (jnp.dot(out1, wih_ref[...], preferred_element_type=_f32)
                   + bih_ref[...])


def _gi_call(s0, s1, h1, dinv, b1r, wihT, bihr):
    return pl.pallas_call(
        _gi_body,
        grid=(NBLK,),
        in_specs=[
            pl.BlockSpec((RB, HALF), lambda i: (i, 0)),
            pl.BlockSpec((RB, HALF), lambda i: (i, 0)),
            pl.BlockSpec((RB, DD), lambda i: (i, 0)),
            pl.BlockSpec((RB, 1), lambda i: (i, 0)),
            pl.BlockSpec((1, DD), lambda i: (0, 0)),
            pl.BlockSpec((DD, 3 * DD), lambda i: (0, 0)),
            pl.BlockSpec((1, 3 * DD), lambda i: (0, 0)),
        ],
        out_specs=pl.BlockSpec((RB, 3 * DD), lambda i: (i, 0)),
        out_shape=jax.ShapeDtypeStruct((NP, 3 * DD), _f32),
        compiler_params=pltpu.CompilerParams(
            dimension_semantics=("arbitrary",)),
    )(s0, s1, h1, dinv, b1r, wihT, bihr)


def _gru_body(gi_ref, whh_ref, bhn_ref, w2_ref, dinv_ref, q_ref,
              h_scr, ys_scr):
    i = pl.program_id(0)

    @pl.when(i == 0)
    def _():
        h_scr[...] = jnp.zeros((1, DD), _f32)

    whh = whh_ref[...]
    bhn = bhn_ref[...]

    def step(t, h):
        gi = gi_ref[pl.ds(t, 1), :]
        gh = jnp.dot(h.astype(jnp.bfloat16), whh, preferred_element_type=_f32)
        r = 0.5 * jnp.tanh(0.5 * (gi[:, :DD] + gh[:, :DD])) + 0.5
        z = 0.5 * jnp.tanh(0.5 * (gi[:, DD:2 * DD] + gh[:, DD:2 * DD])) + 0.5
        ng = jnp.tanh(gi[:, 2 * DD:] + r * (gh[:, 2 * DD:] + bhn))
        h = ng + z * (h - ng)
        ys_scr[pl.ds(t, 1), :] = h
        return h

    def step32(t32, h):
        t = 32 * t32
        for u in range(32):
            h = step(t + u, h)
        return h

    h = lax.fori_loop(0, RB // 32, step32, h_scr[...])
    h_scr[...] = h
    yb = jnp.dot(ys_scr[...], w2_ref[...], preferred_element_type=_f32)
    q_ref[...] = dinv_ref[...] * yb


def _gru_call(gi, whhT_bf, bhn, w2p, dinv):
    return pl.pallas_call(
        _gru_body,
        grid=(NBLK,),
        in_specs=[
            pl.BlockSpec((RB, 3 * DD), lambda i: (i, 0)),
            pl.BlockSpec((DD, 3 * DD), lambda i: (0, 0)),
            pl.BlockSpec((1, DD), lambda i: (0, 0)),
            pl.BlockSpec((DD, HALF), lambda i: (0, 0)),
            pl.BlockSpec((RB, 1), lambda i: (i, 0)),
        ],
        out_specs=pl.BlockSpec((RB, HALF), lambda i: (i, 0)),
        out_shape=jax.ShapeDtypeStruct((NP, HALF), _f32),
        scratch_shapes=[pltpu.VMEM((1, DD), _f32),
                        pltpu.VMEM((RB, DD), _f32)],
        compiler_params=pltpu.CompilerParams(
            dimension_semantics=("arbitrary",)),
    )(gi, whhT_bf, bhn, w2p, dinv)


def _fin_body(q_hbm, src_hbm, dst_hbm, dinv_hbm, b2_hbm, z1_hbm, out_hbm,
              src_v, dst_v, vals_v, accv, qv, dv, outv, b2v, acc_sp, sem):
    c = lax.axis_index("c")
    s = lax.axis_index("s")

    @pl.when(c == 0)
    def _():
        @pl.when(s == 0)
        def _():
            pltpu.sync_copy(z1_hbm, acc_sp)

        plsc.subcore_barrier()
        pltpu.sync_copy(src_hbm.at[s], src_v)
        pltpu.sync_copy(dst_hbm.at[s], dst_v)

        def body(j, carry):
            pltpu.async_copy(q_hbm.at[src_v.at[j]], vals_v, sem).wait()
            pltpu.sync_copy(vals_v, acc_sp.at[dst_v.at[j]], add=True)
            return carry

        lax.fori_loop(0, 80, body, 0)
        plsc.subcore_barrier()
        rows = pl.ds(s * (NP // 16), NP // 16)
        pltpu.sync_copy(acc_sp.at[rows], accv)
        pltpu.sync_copy(q_hbm.at[rows], qv)
        pltpu.sync_copy(dinv_hbm.at[rows], dv)
        pltpu.sync_copy(b2_hbm, b2v)
        b2x = b2v[...]
        for k in range(NP // 16 // 16):
            sl = pl.ds(k * 16, 16)
            outv[sl] = dv[sl] * (accv[sl] + qv[sl]) + b2x
        pltpu.sync_copy(outv, out_hbm.at[rows])


def _fin_call(q1d, src16, dst16, dinv1d, b2v16, z1):
    f = pl.kernel(
        _fin_body,
        out_type=jax.ShapeDtypeStruct((NP,), _f32),
        mesh=plsc.VectorSubcoreMesh(core_axis_name="c", subcore_axis_name="s", num_cores=2, num_subcores=16),
        scratch_types=[pltpu.VMEM((80, 128), jnp.int32),
                       pltpu.VMEM((80, 128), jnp.int32),
                       pltpu.VMEM((128,), _f32),
                       pltpu.VMEM((NP // 16,), _f32),
                       pltpu.VMEM((NP // 16,), _f32),
                       pltpu.VMEM((NP // 16,), _f32),
                       pltpu.VMEM((NP // 16,), _f32),
                       pltpu.VMEM((16,), _f32),
                       pltpu.VMEM_SHARED((NP,), _f32),
                       pltpu.SemaphoreType.DMA],
    )
    return f(q1d, src16, dst16, dinv1d, b2v16, z1)


def kernel(x, edge_index, W1, b1, W_ih, W_hh, b_ih, b_hh, W2, b2):
    src = edge_index[0].astype(jnp.int32)
    dst = edge_index[1].astype(jnp.int32)
    pad = jnp.full((EP - EE,), PAD_IDX, jnp.int32)
    src_p = jnp.concatenate([src, pad])
    dst_p = jnp.concatenate([dst, pad])
    src32 = src_p.reshape(32, 40, 128)
    dst32 = dst_p.reshape(32, 40, 128)
    src16 = src_p.reshape(16, 80, 128)
    dst16 = dst_p.reshape(16, 80, 128)

    xp = jnp.zeros((NP, DD), _f32).at[:NN].set(x)
    z1 = jnp.zeros((NP,), _f32)
    z2 = jnp.zeros((NP, HALF), _f32)
    w2p = jnp.zeros((DD, HALF), _f32).at[:, 0].set(W2[:, 0])
    b2v16 = jnp.broadcast_to(b2, (16,)).astype(_f32)

    d0, d1 = _deg_call(dst32, z1)
    h1, g0, g1, dinv = _mm1_call(xp, W1, d0.reshape(NP, 1), d1.reshape(NP, 1))
    bias_rz = jnp.concatenate([b_hh[:2 * DD], jnp.zeros((DD,), _f32)])
    s0, s1 = _mp_call(g0, g1, src_p.reshape(16, NCHK, CHK),
                      dst_p.reshape(16, NCHK, CHK), z2)
    gi = _gi_call(s0, s1, h1, dinv, b1.reshape(1, DD), W_ih.T,
                  (b_ih + bias_rz).reshape(1, 3 * DD))
    q2d = _gru_call(gi, W_hh.T.astype(jnp.bfloat16),
                    b_hh[2 * DD:].reshape(1, DD), w2p, dinv)
    q1d = q2d[:, 0]
    out = _fin_call(q1d, src16, dst16, dinv.reshape(NP), b2v16, z1)
    return out[:NN]
